# baseline (speedup 1.0000x reference)
import dataclasses
import functools

import jax
import jax.numpy as jnp
from jax import lax
from jax.experimental import pallas as pl
from jax.experimental.pallas import tpu as pltpu
from jax.experimental.pallas import tpu_sc as plsc

N = 10000
D = 128
E = 320000

NC = 2
NS = 16
NW = NC * NS
K = 96
NCHUNK = 105
EPT = NCHUNK * K
EPAD = NW * EPT
N_ACC = 10240
TRASH = N
RPT = 624
RLAST = N - 15 * RPT

_mesh = plsc.VectorSubcoreMesh(core_axis_name="c", subcore_axis_name="s")


EPT_DEG = E // NW
DCH = 2000

_no_layout_cp = pltpu.CompilerParams()
if "needs_layout_passes" in pltpu.CompilerParams.__dataclass_fields__:
    _no_layout_cp = dataclasses.replace(_no_layout_cp, needs_layout_passes=False)


@functools.partial(
    pl.kernel,
    out_type=jax.ShapeDtypeStruct((NW, N), jnp.float32),
    mesh=_mesh,
    scratch_types=[
        pltpu.VMEM((DCH,), jnp.int32),
        pltpu.VMEM((N,), jnp.float32),
    ],
    compiler_params=_no_layout_cp,
)
def _deg_kernel(dst_hbm, out_hbm, dst_v, cnt_v):
    c = lax.axis_index("c")
    s = lax.axis_index("s")
    wid = s * NC + c
    ones = jnp.ones((16,), jnp.float32)

    @pl.loop(0, N // 16)
    def _zero(j):
        cnt_v[pl.ds(j * 16, 16)] = jnp.zeros((16,), jnp.float32)

    @pl.loop(0, EPT_DEG // DCH)
    def _outer(ic):
        pltpu.sync_copy(dst_hbm.at[pl.ds(wid * EPT_DEG + ic * DCH, DCH)], dst_v)

        @pl.loop(0, DCH // 16)
        def _count(j):
            idx = dst_v[pl.ds(j * 16, 16)]
            plsc.addupdate_scatter(cnt_v, [idx], ones)

    pltpu.sync_copy(cnt_v, out_hbm.at[wid])


@functools.partial(
    pl.kernel,
    out_type=jax.ShapeDtypeStruct((NC, N, D), jnp.float32),
    mesh=_mesh,
    scratch_types=[
        pltpu.VMEM((EPT,), jnp.int32),
        pltpu.VMEM((NCHUNK, K), jnp.int32),
        pltpu.VMEM((K, D), jnp.float32),
        pltpu.VMEM((K, D), jnp.float32),
        pltpu.VMEM_SHARED((N_ACC, D), jnp.float32),
        pltpu.SemaphoreType.DMA,
        pltpu.SemaphoreType.DMA,
        pltpu.SemaphoreType.DMA,
        pltpu.SemaphoreType.DMA,
    ],
)
def _agg_kernel(y_hbm, src_hbm, dst_hbm, out_hbm,
                src_v, dst_v, rows0_v, rows1_v, acc_sh,
                sem0, sem1, ssem0, ssem1):
    c = lax.axis_index("c")
    s = lax.axis_index("s")
    wid = s * NC + c

    pltpu.async_copy(src_hbm.at[pl.ds(wid * EPT, EPT)], src_v, sem0)
    pltpu.async_copy(dst_hbm.at[wid], dst_v, sem1)

    @pl.loop(0, K)
    def _fill_zeros(r):
        @pl.loop(0, D // 16)
        def _fill_cols(c0):
            rows0_v[r, pl.ds(c0 * 16, 16)] = jnp.zeros((16,), jnp.float32)

    @pl.loop(0, RPT // K)
    def _zero_acc(j):
        pltpu.sync_copy(rows0_v, acc_sh.at[pl.ds(s * RPT + j * K, K)])

    @pl.when(s < NS - 1)
    def _zero_tail():
        pltpu.sync_copy(rows0_v.at[pl.ds(0, RPT - (RPT // K) * K)],
                        acc_sh.at[pl.ds(s * RPT + (RPT // K) * K,
                                        RPT - (RPT // K) * K)])

    @pl.when(s == NS - 1)
    def _zero_tail_last():
        pltpu.sync_copy(rows0_v, acc_sh.at[pl.ds(15 * RPT + (RPT // K) * K, K)])

    pltpu.make_async_copy(src_hbm.at[pl.ds(wid * EPT, EPT)], src_v, sem0).wait()
    pltpu.make_async_copy(dst_hbm.at[wid], dst_v, sem1).wait()
    plsc.subcore_barrier()

    def _gather(i, buf, sem):
        return pltpu.async_copy(y_hbm.at[src_v.at[pl.ds(i * K, K)]], buf, sem)

    def _gwait(i, buf, sem):
        pltpu.make_async_copy(y_hbm.at[src_v.at[pl.ds(i * K, K)]],
                              buf, sem).wait()

    def _sstart(i, buf, sem):
        pltpu.async_copy(buf, acc_sh.at[dst_v.at[i]], sem, add=True)

    def _swait(i, buf, sem):
        pltpu.make_async_copy(buf, acc_sh.at[dst_v.at[i]], sem).wait()

    _gather(0, rows0_v, sem0)
    _gather(1, rows1_v, sem1)

    @pl.loop(0, (NCHUNK + 1) // 2)
    def _aggregate(it):
        i = it * 2
        _gwait(i, rows0_v, sem0)
        _sstart(i, rows0_v, ssem0)

        @pl.when(i + 1 < NCHUNK)
        def _odd_drain():
            _gwait(i + 1, rows1_v, sem1)
            _sstart(i + 1, rows1_v, ssem1)

        _swait(i, rows0_v, ssem0)

        @pl.when(i + 2 < NCHUNK)
        def _next_even():
            _gather(i + 2, rows0_v, sem0)

        @pl.when(i + 1 < NCHUNK)
        def _odd_done():
            _swait(i + 1, rows1_v, ssem1)

            @pl.when(i + 3 < NCHUNK)
            def _next_odd():
                _gather(i + 3, rows1_v, sem1)

    plsc.subcore_barrier()

    @pl.when(s < NS - 1)
    def _out_main():
        pltpu.sync_copy(acc_sh.at[pl.ds(s * RPT, RPT)],
                        out_hbm.at[c, pl.ds(s * RPT, RPT)])

    @pl.when(s == NS - 1)
    def _out_last():
        pltpu.sync_copy(acc_sh.at[pl.ds(15 * RPT, RLAST)],
                        out_hbm.at[c, pl.ds(15 * RPT, RLAST)])


_BLK = 2000


def _rsqrt_deg(degt_ref):
    return lax.rsqrt(1.0 + jnp.sum(degt_ref[...], axis=1, keepdims=True))


def _tca_body(x_ref, w_ref, xw_ref):
    xw_ref[...] = jnp.dot(x_ref[...], w_ref[...],
                          preferred_element_type=jnp.float32)


def _tc1b_body(degt_ref, xw_ref, y_ref):
    y_ref[...] = xw_ref[...] * _rsqrt_deg(degt_ref)


def _tc2_body(degt_ref, aggp_ref, y1_ref, w_ref, b1_ref, y2_ref):
    d = _rsqrt_deg(degt_ref)
    pre = d * (aggp_ref[0] + aggp_ref[1] + y1_ref[...]) + b1_ref[...]
    h = jnp.maximum(pre, 0.0)
    hw = jnp.dot(h, w_ref[...], preferred_element_type=jnp.float32)
    y2_ref[...] = hw * d


def _tc3_body(degt_ref, aggp_ref, y2_ref, b2_ref, o_ref):
    d = _rsqrt_deg(degt_ref)
    o_ref[...] = (d * (aggp_ref[0] + aggp_ref[1] + y2_ref[...])
                  + b2_ref[...])


def _row_spec():
    return pl.BlockSpec((_BLK, D), lambda i: (i, 0))


def _degt_spec():
    return pl.BlockSpec((_BLK, NW), lambda i: (i, 0))


def _tca(x, W1):
    return pl.pallas_call(
        _tca_body,
        grid=(N // _BLK,),
        in_specs=[_row_spec(), pl.BlockSpec((D, D), lambda i: (0, 0))],
        out_specs=_row_spec(),
        out_shape=jax.ShapeDtypeStruct((N, D), jnp.float32),
    )(x, W1)


def _tc1b(degt, xw):
    return pl.pallas_call(
        _tc1b_body,
        grid=(N // _BLK,),
        in_specs=[_degt_spec(), _row_spec()],
        out_specs=_row_spec(),
        out_shape=jax.ShapeDtypeStruct((N, D), jnp.float32),
    )(degt, xw)


def _tc2(degt, aggp, y1, W2, b1):
    return pl.pallas_call(
        _tc2_body,
        grid=(N // _BLK,),
        in_specs=[
            _degt_spec(),
            pl.BlockSpec((NC, _BLK, D), lambda i: (0, i, 0)),
            _row_spec(),
            pl.BlockSpec((D, D), lambda i: (0, 0)),
            pl.BlockSpec((1, D), lambda i: (0, 0)),
        ],
        out_specs=_row_spec(),
        out_shape=jax.ShapeDtypeStruct((N, D), jnp.float32),
    )(degt, aggp, y1, W2, b1)


def _tc3(degt, aggp, y2, b2):
    return pl.pallas_call(
        _tc3_body,
        grid=(N // _BLK,),
        in_specs=[
            _degt_spec(),
            pl.BlockSpec((NC, _BLK, D), lambda i: (0, i, 0)),
            _row_spec(),
            pl.BlockSpec((1, D), lambda i: (0, 0)),
        ],
        out_specs=_row_spec(),
        out_shape=jax.ShapeDtypeStruct((N, D), jnp.float32),
    )(degt, aggp, y2, b2)


def kernel(x, edge_index, W1, b1, W2, b2):
    src = edge_index[0].astype(jnp.int32)
    dst = edge_index[1].astype(jnp.int32)
    pad = EPAD - E
    src_p = jnp.concatenate([src, jnp.zeros((pad,), jnp.int32)])
    dst_p = jnp.concatenate([dst, jnp.full((pad,), TRASH, jnp.int32)])
    dst3 = dst_p.reshape(NW, NCHUNK, K)
    degp = _deg_kernel(dst)
    xw = _tca(x, W1)
    degt = degp.T
    y1 = _tc1b(degt, xw)
    aggp1 = _agg_kernel(y1, src_p, dst3)
    y2 = _tc2(degt, aggp1, y1, W2, b1.reshape(1, D))
    aggp2 = _agg_kernel(y2, src_p, dst3)
    return _tc3(degt, aggp2, y2, b2.reshape(1, D))


# speedup vs baseline: 1.5083x; 1.5083x over previous
import dataclasses
import functools

import jax
import jax.numpy as jnp
from jax import lax
from jax.experimental import pallas as pl
from jax.experimental.pallas import tpu as pltpu
from jax.experimental.pallas import tpu_sc as plsc

N = 10000
D = 128
E = 320000

NC = 2
NS = 16
NW = NC * NS
K = 80
NCHUNK = 125
EPT = NCHUNK * K
EPAD = NW * EPT
N_ACC = N
TRASH = N
RPT = 624
RLAST = N - 15 * RPT

_mesh = plsc.VectorSubcoreMesh(core_axis_name="c", subcore_axis_name="s")


EPT_DEG = E // NW
DCH = 2000

_no_layout_cp = pltpu.CompilerParams()
if "needs_layout_passes" in pltpu.CompilerParams.__dataclass_fields__:
    _no_layout_cp = dataclasses.replace(_no_layout_cp, needs_layout_passes=False)


@functools.partial(
    pl.kernel,
    out_type=jax.ShapeDtypeStruct((NW, N), jnp.float32),
    mesh=_mesh,
    scratch_types=[
        pltpu.VMEM((DCH,), jnp.int32),
        pltpu.VMEM((N,), jnp.float32),
    ],
    compiler_params=_no_layout_cp,
)
def _deg_kernel(dst_hbm, out_hbm, dst_v, cnt_v):
    c = lax.axis_index("c")
    s = lax.axis_index("s")
    wid = s * NC + c
    ones = jnp.ones((16,), jnp.float32)

    @pl.loop(0, N // 16)
    def _zero(j):
        cnt_v[pl.ds(j * 16, 16)] = jnp.zeros((16,), jnp.float32)

    @pl.loop(0, EPT_DEG // DCH)
    def _outer(ic):
        pltpu.sync_copy(dst_hbm.at[pl.ds(wid * EPT_DEG + ic * DCH, DCH)], dst_v)

        @pl.loop(0, DCH // 16)
        def _count(j):
            idx = dst_v[pl.ds(j * 16, 16)]
            plsc.addupdate_scatter(cnt_v, [idx], ones)

    pltpu.sync_copy(cnt_v, out_hbm.at[wid])


@functools.partial(
    pl.kernel,
    out_type=jax.ShapeDtypeStruct((NC, N, D), jnp.float32),
    mesh=_mesh,
    scratch_types=[
        pltpu.VMEM((EPT,), jnp.int32),
        pltpu.VMEM((NCHUNK, K), jnp.int32),
        pltpu.VMEM((K, D), jnp.float32),
        pltpu.VMEM((K, D), jnp.float32),
        pltpu.VMEM_SHARED((N_ACC, D), jnp.float32),
        pltpu.SemaphoreType.DMA,
        pltpu.SemaphoreType.DMA,
        pltpu.SemaphoreType.DMA,
        pltpu.SemaphoreType.DMA,
    ],
)
def _agg_kernel(y_hbm, src_hbm, dst_hbm, out_hbm,
                src_v, dst_v, rows0_v, rows1_v, acc_sh,
                sem0, sem1, ssem0, ssem1):
    c = lax.axis_index("c")
    s = lax.axis_index("s")
    wid = s * NC + c

    pltpu.async_copy(src_hbm.at[pl.ds(wid * EPT, EPT)], src_v, sem0)
    pltpu.async_copy(dst_hbm.at[wid], dst_v, sem1)

    @pl.loop(0, K)
    def _fill_zeros(r):
        @pl.loop(0, D // 16)
        def _fill_cols(c0):
            rows0_v[r, pl.ds(c0 * 16, 16)] = jnp.zeros((16,), jnp.float32)

    @pl.loop(0, RPT // K)
    def _zero_acc(j):
        pltpu.sync_copy(rows0_v, acc_sh.at[pl.ds(s * RPT + j * K, K)])

    @pl.when(s < NS - 1)
    def _zero_tail():
        pltpu.sync_copy(rows0_v.at[pl.ds(0, RPT - (RPT // K) * K)],
                        acc_sh.at[pl.ds(s * RPT + (RPT // K) * K,
                                        RPT - (RPT // K) * K)])

    @pl.when(s == NS - 1)
    def _zero_tail_last():
        pltpu.sync_copy(rows0_v, acc_sh.at[pl.ds(15 * RPT + (RPT // K) * K, K)])

    pltpu.make_async_copy(src_hbm.at[pl.ds(wid * EPT, EPT)], src_v, sem0).wait()
    pltpu.make_async_copy(dst_hbm.at[wid], dst_v, sem1).wait()
    plsc.subcore_barrier()

    def _gather(i, buf, sem):
        return pltpu.async_copy(y_hbm.at[src_v.at[pl.ds(i * K, K)]], buf, sem)

    def _gwait(i, buf, sem):
        pltpu.make_async_copy(y_hbm.at[src_v.at[pl.ds(i * K, K)]],
                              buf, sem).wait()

    def _sstart(i, buf, sem):
        pltpu.async_copy(buf, acc_sh.at[dst_v.at[i]], sem, add=True)

    def _swait(i, buf, sem):
        pltpu.make_async_copy(buf, acc_sh.at[dst_v.at[i]], sem).wait()

    _gather(0, rows0_v, sem0)
    _gather(1, rows1_v, sem1)

    @pl.loop(0, (NCHUNK + 1) // 2)
    def _aggregate(it):
        i = it * 2
        _gwait(i, rows0_v, sem0)
        _sstart(i, rows0_v, ssem0)

        @pl.when(i + 1 < NCHUNK)
        def _odd_drain():
            _gwait(i + 1, rows1_v, sem1)
            _sstart(i + 1, rows1_v, ssem1)

        _swait(i, rows0_v, ssem0)

        @pl.when(i + 2 < NCHUNK)
        def _next_even():
            _gather(i + 2, rows0_v, sem0)

        @pl.when(i + 1 < NCHUNK)
        def _odd_done():
            _swait(i + 1, rows1_v, ssem1)

            @pl.when(i + 3 < NCHUNK)
            def _next_odd():
                _gather(i + 3, rows1_v, sem1)

    plsc.subcore_barrier()

    @pl.when(s < NS - 1)
    def _out_main():
        pltpu.sync_copy(acc_sh.at[pl.ds(s * RPT, RPT)],
                        out_hbm.at[c, pl.ds(s * RPT, RPT)])

    @pl.when(s == NS - 1)
    def _out_last():
        pltpu.sync_copy(acc_sh.at[pl.ds(15 * RPT, RLAST)],
                        out_hbm.at[c, pl.ds(15 * RPT, RLAST)])


_BLK = 2000


def _rsqrt_deg(degt_ref):
    return lax.rsqrt(1.0 + jnp.sum(degt_ref[...], axis=1, keepdims=True))


def _tca_body(x_ref, w_ref, xw_ref):
    xw_ref[...] = jnp.dot(x_ref[...], w_ref[...],
                          preferred_element_type=jnp.float32)


def _tc1b_body(degt_ref, xw_ref, y_ref):
    y_ref[...] = xw_ref[...] * _rsqrt_deg(degt_ref)


def _tc2_body(degt_ref, aggp_ref, y1_ref, w_ref, b1_ref, y2_ref):
    d = _rsqrt_deg(degt_ref)
    pre = d * (aggp_ref[0] + aggp_ref[1] + y1_ref[...]) + b1_ref[...]
    h = jnp.maximum(pre, 0.0)
    hw = jnp.dot(h, w_ref[...], preferred_element_type=jnp.float32)
    y2_ref[...] = hw * d


def _tc3_body(degt_ref, aggp_ref, y2_ref, b2_ref, o_ref):
    d = _rsqrt_deg(degt_ref)
    o_ref[...] = (d * (aggp_ref[0] + aggp_ref[1] + y2_ref[...])
                  + b2_ref[...])


def _row_spec():
    return pl.BlockSpec((_BLK, D), lambda i: (i, 0))


def _degt_spec():
    return pl.BlockSpec((_BLK, NW), lambda i: (i, 0))


def _tca(x, W1):
    return pl.pallas_call(
        _tca_body,
        grid=(N // _BLK,),
        in_specs=[_row_spec(), pl.BlockSpec((D, D), lambda i: (0, 0))],
        out_specs=_row_spec(),
        out_shape=jax.ShapeDtypeStruct((N, D), jnp.float32),
    )(x, W1)


def _tc1b(degt, xw):
    return pl.pallas_call(
        _tc1b_body,
        grid=(N // _BLK,),
        in_specs=[_degt_spec(), _row_spec()],
        out_specs=_row_spec(),
        out_shape=jax.ShapeDtypeStruct((N, D), jnp.float32),
    )(degt, xw)


def _tc2(degt, aggp, y1, W2, b1):
    return pl.pallas_call(
        _tc2_body,
        grid=(N // _BLK,),
        in_specs=[
            _degt_spec(),
            pl.BlockSpec((NC, _BLK, D), lambda i: (0, i, 0)),
            _row_spec(),
            pl.BlockSpec((D, D), lambda i: (0, 0)),
            pl.BlockSpec((1, D), lambda i: (0, 0)),
        ],
        out_specs=_row_spec(),
        out_shape=jax.ShapeDtypeStruct((N, D), jnp.float32),
    )(degt, aggp, y1, W2, b1)


def _tc3(degt, aggp, y2, b2):
    return pl.pallas_call(
        _tc3_body,
        grid=(N // _BLK,),
        in_specs=[
            _degt_spec(),
            pl.BlockSpec((NC, _BLK, D), lambda i: (0, i, 0)),
            _row_spec(),
            pl.BlockSpec((1, D), lambda i: (0, 0)),
        ],
        out_specs=_row_spec(),
        out_shape=jax.ShapeDtypeStruct((N, D), jnp.float32),
    )(degt, aggp, y2, b2)


def kernel(x, edge_index, W1, b1, W2, b2):
    src = edge_index[0].astype(jnp.int32)
    dst = edge_index[1].astype(jnp.int32)
    pad = EPAD - E
    src_p = jnp.concatenate([src, jnp.zeros((pad,), jnp.int32)])
    dst_p = jnp.concatenate([dst, jnp.full((pad,), TRASH, jnp.int32)])
    dst3 = dst_p.reshape(NW, NCHUNK, K)
    degp = _deg_kernel(dst)
    xw = _tca(x, W1)
    degt = degp.T
    y1 = _tc1b(degt, xw)
    aggp1 = _agg_kernel(y1, src_p, dst3)
    y2 = _tc2(degt, aggp1, y1, W2, b1.reshape(1, D))
    aggp2 = _agg_kernel(y2, src_p, dst3)
    return _tc3(degt, aggp2, y2, b2.reshape(1, D))
